# baseline (speedup 1.0000x reference)
import functools

import jax
import jax.numpy as jnp
from jax import lax
from jax.experimental import pallas as pl
from jax.experimental.pallas import tpu as pltpu

_NCPAD = 128
_BI = 80
_STRIDE = 4
_SLOT_CAP = 30


def _dot(a, b):
    return lax.dot_general(a, b, (((1,), (0,)), ((), ())),
                           preferred_element_type=jnp.float32)


def _body(nclass, nsteps, xb_ref, adj_ref, w1_ref, b1_ref, w4_ref, b4_ref,
          o_ref, g_scr, stash_scr, acc_scr):
    i = pl.program_id(1)
    phase0 = pl.program_id(0) == 0
    hit = (jnp.bitwise_and(i, _STRIDE - 1) == 0) & (i < _SLOT_CAP * _STRIDE)
    srow = pl.multiple_of(lax.shift_right_logical(i, 2) * _BI, _BI)

    @pl.when(phase0)
    def _():
        a = adj_ref[...].astype(jnp.bfloat16)

        @pl.when(hit)
        def _():
            stash_scr[pl.ds(srow, _BI), :] = a

        t = _dot(a, xb_ref[...]).astype(jnp.bfloat16)
        h = jnp.maximum(_dot(t, w1_ref[...]) + b1_ref[...], 0.0)
        g_scr[pl.ds(i * _BI, _BI), :] = _dot(
            h.astype(jnp.bfloat16), w4_ref[...]).astype(jnp.bfloat16)

    @pl.when(jnp.logical_not(phase0) & hit)
    def _():
        acc_scr[...] = _dot(stash_scr[pl.ds(srow, _BI), :], g_scr[...])

    @pl.when(jnp.logical_not(phase0) & jnp.logical_not(hit))
    def _():
        acc_scr[...] = _dot(adj_ref[...].astype(jnp.bfloat16), g_scr[...])

    @pl.when(jnp.logical_not(phase0))
    def _():
        z = acc_scr[...] + b4_ref[...]
        col = lax.broadcasted_iota(jnp.int32, z.shape, 1)
        zm = jnp.where(col < nclass, z, -jnp.inf)
        m = jnp.max(zm, axis=1, keepdims=True)
        lse = jnp.log(jnp.sum(jnp.exp(zm - m), axis=1, keepdims=True))
        o_ref[...] = lax.slice((z - m) - lse, (0, 0), (_BI, nclass))


@jax.jit
def kernel(x, adj, W1, b1, W4, b4):
    n, nfeat = x.shape
    nhid = W1.shape[1]
    nclass = W4.shape[1]
    nsteps = n // _BI
    nslots = min((nsteps + _STRIDE - 1) // _STRIDE, _SLOT_CAP)

    xb = x.astype(jnp.bfloat16)
    w1b = W1.astype(jnp.bfloat16)
    w4b = jnp.pad(W4, ((0, 0), (0, _NCPAD - nclass))).astype(jnp.bfloat16)
    b1r = b1.reshape(1, nhid)
    b4r = jnp.pad(b4, (0, _NCPAD - nclass)).reshape(1, _NCPAD)

    def adj_imap(p, i):
        hit = (jnp.bitwise_and(i, _STRIDE - 1) == 0) & (i < _SLOT_CAP * _STRIDE)
        i1 = jnp.where(hit, jnp.where(i + 1 < nsteps, i + 1, i - 1), i)
        return (jnp.where(p == 0, i, i1), 0)

    return pl.pallas_call(
        functools.partial(_body, nclass, nsteps),
        grid=(2, nsteps),
        in_specs=[
            pl.BlockSpec((n, nfeat), lambda p, i: (0, 0)),
            pl.BlockSpec((_BI, n), adj_imap),
            pl.BlockSpec((nfeat, nhid), lambda p, i: (0, 0)),
            pl.BlockSpec((1, nhid), lambda p, i: (0, 0)),
            pl.BlockSpec((nhid, _NCPAD), lambda p, i: (0, 0)),
            pl.BlockSpec((1, _NCPAD), lambda p, i: (0, 0)),
        ],
        out_specs=pl.BlockSpec(
            (_BI, nclass), lambda p, i: (jnp.where(p == 0, 0, i), 0)),
        out_shape=jax.ShapeDtypeStruct((n, nclass), jnp.float32),
        scratch_shapes=[
            pltpu.VMEM((n, _NCPAD), jnp.bfloat16),
            pltpu.VMEM((nslots * _BI, n), jnp.bfloat16),
            pltpu.VMEM((_BI, _NCPAD), jnp.float32),
        ],
        compiler_params=pltpu.CompilerParams(
            dimension_semantics=("arbitrary", "arbitrary"),
            vmem_limit_bytes=64 * 1024 * 1024),
    )(xb, adj, w1b, b1r, w4b, b4r)


# speedup vs baseline: 1.0126x; 1.0126x over previous
import functools

import jax
import jax.numpy as jnp
from jax import lax
from jax.experimental import pallas as pl
from jax.experimental.pallas import tpu as pltpu

_NCPAD = 128
_BI = 80
_STRIDE = 4
_SLOT_CAP = 0


def _dot(a, b):
    return lax.dot_general(a, b, (((1,), (0,)), ((), ())),
                           preferred_element_type=jnp.float32)


def _body(nclass, nsteps, xb_ref, adj_ref, w1_ref, b1_ref, w4_ref, b4_ref,
          o_ref, g_scr, stash_scr, acc_scr):
    i = pl.program_id(1)
    phase0 = pl.program_id(0) == 0
    hit = (jnp.bitwise_and(i, _STRIDE - 1) == 0) & (i < _SLOT_CAP * _STRIDE)
    srow = pl.multiple_of(lax.shift_right_logical(i, 2) * _BI, _BI)

    @pl.when(phase0)
    def _():
        a = adj_ref[...].astype(jnp.bfloat16)

        @pl.when(hit)
        def _():
            stash_scr[pl.ds(srow, _BI), :] = a

        t = _dot(a, xb_ref[...]).astype(jnp.bfloat16)
        h = jnp.maximum(_dot(t, w1_ref[...]) + b1_ref[...], 0.0)
        g_scr[pl.ds(i * _BI, _BI), :] = _dot(
            h.astype(jnp.bfloat16), w4_ref[...]).astype(jnp.bfloat16)

    @pl.when(jnp.logical_not(phase0) & hit)
    def _():
        acc_scr[...] = _dot(stash_scr[pl.ds(srow, _BI), :], g_scr[...])

    @pl.when(jnp.logical_not(phase0) & jnp.logical_not(hit))
    def _():
        acc_scr[...] = _dot(adj_ref[...].astype(jnp.bfloat16), g_scr[...])

    @pl.when(jnp.logical_not(phase0))
    def _():
        z = acc_scr[...] + b4_ref[...]
        col = lax.broadcasted_iota(jnp.int32, z.shape, 1)
        zm = jnp.where(col < nclass, z, -jnp.inf)
        m = jnp.max(zm, axis=1, keepdims=True)
        lse = jnp.log(jnp.sum(jnp.exp(zm - m), axis=1, keepdims=True))
        o_ref[...] = lax.slice((z - m) - lse, (0, 0), (_BI, nclass))


@jax.jit
def kernel(x, adj, W1, b1, W4, b4):
    n, nfeat = x.shape
    nhid = W1.shape[1]
    nclass = W4.shape[1]
    nsteps = n // _BI
    nslots = max(1, min((nsteps + _STRIDE - 1) // _STRIDE, _SLOT_CAP))

    xb = x.astype(jnp.bfloat16)
    w1b = W1.astype(jnp.bfloat16)
    w4b = jnp.pad(W4, ((0, 0), (0, _NCPAD - nclass))).astype(jnp.bfloat16)
    b1r = b1.reshape(1, nhid)
    b4r = jnp.pad(b4, (0, _NCPAD - nclass)).reshape(1, _NCPAD)

    def adj_imap(p, i):
        hit = (jnp.bitwise_and(i, _STRIDE - 1) == 0) & (i < _SLOT_CAP * _STRIDE)
        i1 = jnp.where(hit, jnp.where(i + 1 < nsteps, i + 1, i - 1), i)
        return (jnp.where(p == 0, i, i1), 0)

    return pl.pallas_call(
        functools.partial(_body, nclass, nsteps),
        grid=(2, nsteps),
        in_specs=[
            pl.BlockSpec((n, nfeat), lambda p, i: (0, 0)),
            pl.BlockSpec((_BI, n), adj_imap),
            pl.BlockSpec((nfeat, nhid), lambda p, i: (0, 0)),
            pl.BlockSpec((1, nhid), lambda p, i: (0, 0)),
            pl.BlockSpec((nhid, _NCPAD), lambda p, i: (0, 0)),
            pl.BlockSpec((1, _NCPAD), lambda p, i: (0, 0)),
        ],
        out_specs=pl.BlockSpec(
            (_BI, nclass), lambda p, i: (jnp.where(p == 0, 0, i), 0)),
        out_shape=jax.ShapeDtypeStruct((n, nclass), jnp.float32),
        scratch_shapes=[
            pltpu.VMEM((n, _NCPAD), jnp.bfloat16),
            pltpu.VMEM((nslots * _BI, n), jnp.bfloat16),
            pltpu.VMEM((_BI, _NCPAD), jnp.float32),
        ],
        compiler_params=pltpu.CompilerParams(
            dimension_semantics=("arbitrary", "arbitrary"),
            vmem_limit_bytes=64 * 1024 * 1024),
    )(xb, adj, w1b, b1r, w4b, b4r)


# speedup vs baseline: 1.5394x; 1.5202x over previous
import functools

import jax
import jax.numpy as jnp
from jax import lax
from jax.experimental import pallas as pl
from jax.experimental.pallas import tpu as pltpu

_NCPAD = 128
_BI = 400
_STRIDE = 4
_SLOT_CAP = 0


def _dot(a, b):
    return lax.dot_general(a, b, (((1,), (0,)), ((), ())),
                           preferred_element_type=jnp.float32)


def _body(nclass, nsteps, xb_ref, adj_ref, w1_ref, b1_ref, w4_ref, b4_ref,
          o_ref, g_scr, stash_scr, acc_scr):
    i = pl.program_id(1)
    phase0 = pl.program_id(0) == 0
    hit = (jnp.bitwise_and(i, _STRIDE - 1) == 0) & (i < _SLOT_CAP * _STRIDE)
    srow = pl.multiple_of(lax.shift_right_logical(i, 2) * _BI, _BI)

    @pl.when(phase0)
    def _():
        a = adj_ref[...].astype(jnp.bfloat16)

        @pl.when(hit)
        def _():
            stash_scr[pl.ds(srow, _BI), :] = a

        t = _dot(a, xb_ref[...]).astype(jnp.bfloat16)
        h = jnp.maximum(_dot(t, w1_ref[...]) + b1_ref[...], 0.0)
        g_scr[pl.ds(i * _BI, _BI), :] = _dot(
            h.astype(jnp.bfloat16), w4_ref[...]).astype(jnp.bfloat16)

    @pl.when(jnp.logical_not(phase0) & hit)
    def _():
        acc_scr[...] = _dot(stash_scr[pl.ds(srow, _BI), :], g_scr[...])

    @pl.when(jnp.logical_not(phase0) & jnp.logical_not(hit))
    def _():
        acc_scr[...] = _dot(adj_ref[...].astype(jnp.bfloat16), g_scr[...])

    @pl.when(jnp.logical_not(phase0))
    def _():
        z = acc_scr[...] + b4_ref[...]
        col = lax.broadcasted_iota(jnp.int32, z.shape, 1)
        zm = jnp.where(col < nclass, z, -jnp.inf)
        m = jnp.max(zm, axis=1, keepdims=True)
        lse = jnp.log(jnp.sum(jnp.exp(zm - m), axis=1, keepdims=True))
        o_ref[...] = lax.slice((z - m) - lse, (0, 0), (_BI, nclass))


@jax.jit
def kernel(x, adj, W1, b1, W4, b4):
    n, nfeat = x.shape
    nhid = W1.shape[1]
    nclass = W4.shape[1]
    nsteps = n // _BI
    nslots = max(1, min((nsteps + _STRIDE - 1) // _STRIDE, _SLOT_CAP))

    xb = x.astype(jnp.bfloat16)
    w1b = W1.astype(jnp.bfloat16)
    w4b = jnp.pad(W4, ((0, 0), (0, _NCPAD - nclass))).astype(jnp.bfloat16)
    b1r = b1.reshape(1, nhid)
    b4r = jnp.pad(b4, (0, _NCPAD - nclass)).reshape(1, _NCPAD)

    def adj_imap(p, i):
        hit = (jnp.bitwise_and(i, _STRIDE - 1) == 0) & (i < _SLOT_CAP * _STRIDE)
        i1 = jnp.where(hit, jnp.where(i + 1 < nsteps, i + 1, i - 1), i)
        return (jnp.where(p == 0, i, i1), 0)

    return pl.pallas_call(
        functools.partial(_body, nclass, nsteps),
        grid=(2, nsteps),
        in_specs=[
            pl.BlockSpec((n, nfeat), lambda p, i: (0, 0)),
            pl.BlockSpec((_BI, n), adj_imap),
            pl.BlockSpec((nfeat, nhid), lambda p, i: (0, 0)),
            pl.BlockSpec((1, nhid), lambda p, i: (0, 0)),
            pl.BlockSpec((nhid, _NCPAD), lambda p, i: (0, 0)),
            pl.BlockSpec((1, _NCPAD), lambda p, i: (0, 0)),
        ],
        out_specs=pl.BlockSpec(
            (_BI, nclass), lambda p, i: (jnp.where(p == 0, 0, i), 0)),
        out_shape=jax.ShapeDtypeStruct((n, nclass), jnp.float32),
        scratch_shapes=[
            pltpu.VMEM((n, _NCPAD), jnp.bfloat16),
            pltpu.VMEM((nslots * _BI, n), jnp.bfloat16),
            pltpu.VMEM((_BI, _NCPAD), jnp.float32),
        ],
        compiler_params=pltpu.CompilerParams(
            dimension_semantics=("arbitrary", "arbitrary"),
            vmem_limit_bytes=64 * 1024 * 1024),
    )(xb, adj, w1b, b1r, w4b, b4r)
